# Initial kernel scaffold; baseline (speedup 1.0000x reference)
import functools
import math

import jax
import jax.numpy as jnp
from jax import lax
from jax.experimental import pallas as pl
from jax.experimental.pallas import tpu as pltpu
from jax.experimental.pallas import tpu_sc as plsc

NC = 2
NS = 16
LK = 128

_BN_C = (1.0 + 1e-5) ** -0.5


def _seg_rows(nrows, w, ch):
    mesh = plsc.VectorSubcoreMesh(core_axis_name="c", subcore_axis_name="s",
                                  num_cores=NC, num_subcores=NS)
    rpt = nrows // NS

    @functools.partial(
        pl.kernel,
        out_type=jax.ShapeDtypeStruct((NC, nrows, w), jnp.float32),
        mesh=mesh,
        scratch_types=[
            pltpu.VMEM((ch, LK), jnp.int32),
            pltpu.VMEM((ch, LK), jnp.int32),
            pltpu.VMEM((LK, w), jnp.float32),
            pltpu.VMEM((LK, w), jnp.float32),
            pltpu.VMEM_SHARED((nrows, w), jnp.float32),
            pltpu.SemaphoreType.DMA,
            pltpu.SemaphoreType.DMA,
        ],
    )
    def k(table_h, src_h, dst_h, z_h, out_h, src_v, dst_v, r0, r1, acc, s0, s1):
        c = lax.axis_index("c")
        s = lax.axis_index("s")
        rows = (r0, r1)
        sems = (s0, s1)
        pltpu.sync_copy(src_h.at[c, s], src_v)
        pltpu.sync_copy(dst_h.at[c, s], dst_v)
        pltpu.async_copy(table_h.at[src_v.at[0]], r0, s0)
        pltpu.async_copy(table_h.at[src_v.at[1]], r1, s1)
        pltpu.sync_copy(z_h, acc.at[pl.ds(s * rpt, rpt)])
        plsc.subcore_barrier()

        @pl.loop(0, ch, step=2)
        def _(j):
            for b in range(2):
                cidx = j + b
                pltpu.make_async_copy(table_h.at[src_v.at[cidx]], rows[b],
                                      sems[b]).wait()
                pltpu.sync_copy(rows[b], acc.at[dst_v.at[cidx]], add=True)

                @pl.when(cidx + 2 < ch)
                def _():
                    pltpu.async_copy(table_h.at[src_v.at[cidx + 2]], rows[b],
                                     sems[b])

        plsc.subcore_barrier()
        pltpu.sync_copy(acc.at[pl.ds(s * rpt, rpt)],
                        out_h.at[c, pl.ds(s * rpt, rpt)])

    return k


def _deg_rows(nrows, w, ch):
    mesh = plsc.VectorSubcoreMesh(core_axis_name="c", subcore_axis_name="s",
                                  num_cores=NC, num_subcores=NS)
    rpt = nrows // NS

    @functools.partial(
        pl.kernel,
        out_type=jax.ShapeDtypeStruct((NC, nrows, w), jnp.float32),
        mesh=mesh,
        scratch_types=[
            pltpu.VMEM((ch, LK), jnp.int32),
            pltpu.VMEM((LK, w), jnp.float32),
            pltpu.VMEM_SHARED((nrows, w), jnp.float32),
        ],
    )
    def k(dst_h, ones_h, z_h, out_h, dst_v, ones_v, acc):
        c = lax.axis_index("c")
        s = lax.axis_index("s")
        pltpu.sync_copy(dst_h.at[c, s], dst_v)
        pltpu.sync_copy(ones_h, ones_v)
        pltpu.sync_copy(z_h, acc.at[pl.ds(s * rpt, rpt)])
        plsc.subcore_barrier()

        @pl.loop(0, ch)
        def _(j):
            pltpu.sync_copy(ones_v, acc.at[dst_v.at[j]], add=True)

        plsc.subcore_barrier()
        pltpu.sync_copy(acc.at[pl.ds(s * rpt, rpt)],
                        out_h.at[c, pl.ds(s * rpt, rpt)])

    return k


def _enc_body(x_ref, w_ref, b_ref, h_ref):
    h_ref[...] = jnp.maximum(
        jnp.dot(x_ref[...], w_ref[...], preferred_element_type=jnp.float32)
        + b_ref[...], 0.0)


def _pre0_body(deg_ref, h_ref, w_ref, dis_ref, hw_ref, *, n):
    deg = deg_ref[0, :n, 0:1] + deg_ref[1, :n, 0:1] + 1.0
    dis = lax.rsqrt(deg)
    dis_ref[...] = dis
    hw_ref[...] = jnp.dot(h_ref[...], w_ref[...],
                          preferred_element_type=jnp.float32) * dis


def _mid_body(sp_ref, hwp_ref, dis_ref, hres_ref, bc_ref, g_ref, be_ref,
              wn_ref, h_ref, hwn_ref, *, n, residual):
    dis = dis_ref[...]
    ssum = sp_ref[0, :n, :] + sp_ref[1, :n, :] + hwp_ref[...]
    agg = ssum * dis + bc_ref[...]
    h = jnp.maximum(g_ref[...] * (agg * _BN_C) + be_ref[...], 0.0)
    if residual:
        h = h + hres_ref[...]
    h_ref[...] = h
    hwn_ref[...] = jnp.dot(h, wn_ref[...],
                           preferred_element_type=jnp.float32) * dis


def _head_body(sp_ref, hwp_ref, dis_ref, hres_ref, bc_ref, g_ref, be_ref,
               wa1_ref, ba1_ref, wa2_ref, ba2_ref, wk1_ref, bk1_ref,
               wk2_ref, bk2_ref, out_ref, *, n):
    dis = dis_ref[...]
    ssum = sp_ref[0, :n, :] + sp_ref[1, :n, :] + hwp_ref[...]
    agg = ssum * dis + bc_ref[...]
    h = jnp.maximum(g_ref[...] * (agg * _BN_C) + be_ref[...], 0.0)
    h = h + hres_ref[...]
    a = jnp.dot(jnp.tanh(jnp.dot(h, wa1_ref[...],
                                 preferred_element_type=jnp.float32)
                         + ba1_ref[...]),
                wa2_ref[...], preferred_element_type=jnp.float32) + ba2_ref[...]
    a = a - jnp.max(a, axis=0, keepdims=True)
    ea = jnp.exp(a)
    a = ea / jnp.sum(ea, axis=0, keepdims=True)
    h = h * a
    out_ref[...] = jnp.dot(
        jnp.maximum(jnp.dot(h, wk1_ref[...],
                            preferred_element_type=jnp.float32)
                    + bk1_ref[...], 0.0),
        wk2_ref[...], preferred_element_type=jnp.float32) + bk2_ref[...]


def _tc(body, out_shape, **kw):
    return pl.pallas_call(functools.partial(body, **kw), out_shape=out_shape)


def kernel(x, edge_index, W_ne, b_ne, Wc0, bc0, g0, be0, Wc1, bc1, g1, be1,
           Wc2, bc2, g2, be2, Wa1, ba1, Wa2, ba2, Wk1, bk1, Wk2, bk2):
    n = x.shape[0]
    h_dim = W_ne.shape[1]
    e = edge_index.shape[1]

    ch = math.ceil(e / (NC * NS * LK))
    ch = ch + (ch % 2)
    epad = NC * NS * ch * LK
    rpt = 8 * math.ceil((n + 8) / (NS * 8))
    nrows = NS * rpt

    src = edge_index[0]
    dst = edge_index[1]
    src_r = jnp.pad(src, (0, epad - e)).reshape(NC, NS, ch, LK)
    dst_r = jnp.pad(dst, (0, epad - e), constant_values=n).reshape(
        NC, NS, ch, LK)

    zrows = jnp.zeros((rpt, h_dim), jnp.float32)
    zdeg = jnp.zeros((rpt, 16), jnp.float32)
    ones16 = jnp.ones((LK, 16), jnp.float32)

    seg = _seg_rows(nrows, h_dim, ch)
    deg_parts = _deg_rows(nrows, 16, ch)(dst_r, ones16, zdeg)

    h0 = _tc(_enc_body, jax.ShapeDtypeStruct((n, h_dim), jnp.float32))(
        x, W_ne, b_ne)

    dis, hw0 = _tc(_pre0_body,
                   (jax.ShapeDtypeStruct((n, 1), jnp.float32),
                    jax.ShapeDtypeStruct((n, h_dim), jnp.float32)),
                   n=n)(deg_parts, h0, Wc0)

    s0 = seg(hw0, src_r, dst_r, zrows)
    h1, hw1 = _tc(_mid_body,
                  (jax.ShapeDtypeStruct((n, h_dim), jnp.float32),
                   jax.ShapeDtypeStruct((n, h_dim), jnp.float32)),
                  n=n, residual=False)(s0, hw0, dis, h0, bc0, g0, be0, Wc1)

    s1 = seg(hw1, src_r, dst_r, zrows)
    h2, hw2 = _tc(_mid_body,
                  (jax.ShapeDtypeStruct((n, h_dim), jnp.float32),
                   jax.ShapeDtypeStruct((n, h_dim), jnp.float32)),
                  n=n, residual=True)(s1, hw1, dis, h1, bc1, g1, be1, Wc2)

    s2 = seg(hw2, src_r, dst_r, zrows)
    out = _tc(_head_body, jax.ShapeDtypeStruct((n, 2), jnp.float32), n=n)(
        s2, hw2, dis, h2, bc2, g2, be2,
        Wa1, ba1, Wa2, ba2, Wk1, bk1, Wk2, bk2)
    return out


# speedup vs baseline: 11.7101x; 11.7101x over previous
import functools
import math

import jax
import jax.numpy as jnp
from jax import lax
from jax.experimental import pallas as pl
from jax.experimental.pallas import tpu as pltpu
from jax.experimental.pallas import tpu_sc as plsc

NC = 2
NS = 16
LK = 128

_BN_C = (1.0 + 1e-5) ** -0.5
_SC_PARAMS = pltpu.CompilerParams(use_tc_tiling_on_sc=False)


def _seg_rows(nrows, w, ch):
    mesh = plsc.VectorSubcoreMesh(core_axis_name="c", subcore_axis_name="s",
                                  num_cores=NC, num_subcores=NS)
    rpt = nrows // NS

    @functools.partial(
        pl.kernel,
        out_type=jax.ShapeDtypeStruct((NC, nrows, w), jnp.float32),
        mesh=mesh,
        compiler_params=_SC_PARAMS,
        scratch_types=[
            pltpu.VMEM((ch, LK), jnp.int32),
            pltpu.VMEM((ch, LK), jnp.int32),
            pltpu.VMEM((LK, w), jnp.float32),
            pltpu.VMEM((LK, w), jnp.float32),
            pltpu.VMEM_SHARED((nrows, w), jnp.float32),
            pltpu.SemaphoreType.DMA,
            pltpu.SemaphoreType.DMA,
        ],
    )
    def k(table_h, src_h, dst_h, z_h, out_h, src_v, dst_v, r0, r1, acc, s0, s1):
        c = lax.axis_index("c")
        s = lax.axis_index("s")
        rows = (r0, r1)
        sems = (s0, s1)
        pltpu.sync_copy(src_h.at[c, s], src_v)
        pltpu.sync_copy(dst_h.at[c, s], dst_v)
        pltpu.async_copy(table_h.at[src_v.at[0]], r0, s0)
        pltpu.async_copy(table_h.at[src_v.at[1]], r1, s1)
        pltpu.sync_copy(z_h, acc.at[pl.ds(s * rpt, rpt)])
        plsc.subcore_barrier()

        @pl.loop(0, ch, step=2)
        def _(j):
            for b in range(2):
                cidx = j + b
                pltpu.make_async_copy(table_h.at[src_v.at[cidx]], rows[b],
                                      sems[b]).wait()
                pltpu.sync_copy(rows[b], acc.at[dst_v.at[cidx]], add=True)

                @pl.when(cidx + 2 < ch)
                def _():
                    pltpu.async_copy(table_h.at[src_v.at[cidx + 2]], rows[b],
                                     sems[b])

        plsc.subcore_barrier()
        pltpu.sync_copy(acc.at[pl.ds(s * rpt, rpt)],
                        out_h.at[c, pl.ds(s * rpt, rpt)])

    return k


def _deg_rows(nrows, w, ch):
    mesh = plsc.VectorSubcoreMesh(core_axis_name="c", subcore_axis_name="s",
                                  num_cores=NC, num_subcores=NS)
    rpt = nrows // NS

    @functools.partial(
        pl.kernel,
        out_type=jax.ShapeDtypeStruct((NC, nrows, w), jnp.float32),
        mesh=mesh,
        compiler_params=_SC_PARAMS,
        scratch_types=[
            pltpu.VMEM((ch, LK), jnp.int32),
            pltpu.VMEM((LK, w), jnp.float32),
            pltpu.VMEM_SHARED((nrows, w), jnp.float32),
        ],
    )
    def k(dst_h, ones_h, z_h, out_h, dst_v, ones_v, acc):
        c = lax.axis_index("c")
        s = lax.axis_index("s")
        pltpu.sync_copy(dst_h.at[c, s], dst_v)
        pltpu.sync_copy(ones_h, ones_v)
        pltpu.sync_copy(z_h, acc.at[pl.ds(s * rpt, rpt)])
        plsc.subcore_barrier()

        @pl.loop(0, ch)
        def _(j):
            pltpu.sync_copy(ones_v, acc.at[dst_v.at[j]], add=True)

        plsc.subcore_barrier()
        pltpu.sync_copy(acc.at[pl.ds(s * rpt, rpt)],
                        out_h.at[c, pl.ds(s * rpt, rpt)])

    return k


def _enc_body(x_ref, w_ref, b_ref, h_ref):
    h_ref[...] = jnp.maximum(
        jnp.dot(x_ref[...], w_ref[...], preferred_element_type=jnp.float32)
        + b_ref[...], 0.0)


def _pre0_body(deg_ref, h_ref, w_ref, dis_ref, hw_ref, *, n):
    deg = deg_ref[0, :n, 0:1] + deg_ref[1, :n, 0:1] + 1.0
    dis = lax.rsqrt(deg)
    dis_ref[...] = dis
    hw_ref[...] = jnp.dot(h_ref[...], w_ref[...],
                          preferred_element_type=jnp.float32) * dis


def _mid_body(sp_ref, hwp_ref, dis_ref, hres_ref, bc_ref, g_ref, be_ref,
              wn_ref, h_ref, hwn_ref, *, n, residual):
    dis = dis_ref[...]
    ssum = sp_ref[0, :n, :] + sp_ref[1, :n, :] + hwp_ref[...]
    agg = ssum * dis + bc_ref[...]
    h = jnp.maximum(g_ref[...] * (agg * _BN_C) + be_ref[...], 0.0)
    if residual:
        h = h + hres_ref[...]
    h_ref[...] = h
    hwn_ref[...] = jnp.dot(h, wn_ref[...],
                           preferred_element_type=jnp.float32) * dis


def _head_body(sp_ref, hwp_ref, dis_ref, hres_ref, bc_ref, g_ref, be_ref,
               wa1_ref, ba1_ref, wa2_ref, ba2_ref, wk1_ref, bk1_ref,
               wk2_ref, bk2_ref, out_ref, *, n):
    dis = dis_ref[...]
    ssum = sp_ref[0, :n, :] + sp_ref[1, :n, :] + hwp_ref[...]
    agg = ssum * dis + bc_ref[...]
    h = jnp.maximum(g_ref[...] * (agg * _BN_C) + be_ref[...], 0.0)
    h = h + hres_ref[...]
    a = jnp.dot(jnp.tanh(jnp.dot(h, wa1_ref[...],
                                 preferred_element_type=jnp.float32)
                         + ba1_ref[...]),
                wa2_ref[...], preferred_element_type=jnp.float32) + ba2_ref[...]
    a = a - jnp.max(a, axis=0, keepdims=True)
    ea = jnp.exp(a)
    a = ea / jnp.sum(ea, axis=0, keepdims=True)
    h = h * a
    out_ref[...] = jnp.dot(
        jnp.maximum(jnp.dot(h, wk1_ref[...],
                            preferred_element_type=jnp.float32)
                    + bk1_ref[...], 0.0),
        wk2_ref[...], preferred_element_type=jnp.float32) + bk2_ref[...]


def _tc(body, out_shape, **kw):
    return pl.pallas_call(functools.partial(body, **kw), out_shape=out_shape)


def kernel(x, edge_index, W_ne, b_ne, Wc0, bc0, g0, be0, Wc1, bc1, g1, be1,
           Wc2, bc2, g2, be2, Wa1, ba1, Wa2, ba2, Wk1, bk1, Wk2, bk2):
    n = x.shape[0]
    h_dim = W_ne.shape[1]
    e = edge_index.shape[1]

    ch = math.ceil(e / (NC * NS * LK))
    ch = ch + (ch % 2)
    epad = NC * NS * ch * LK
    rpt = 8 * math.ceil((n + 8) / (NS * 8))
    nrows = NS * rpt

    src = edge_index[0]
    dst = edge_index[1]
    src_r = jnp.pad(src, (0, epad - e)).reshape(NC, NS, ch, LK)
    dst_r = jnp.pad(dst, (0, epad - e), constant_values=n).reshape(
        NC, NS, ch, LK)

    zrows = jnp.zeros((rpt, h_dim), jnp.float32)
    zdeg = jnp.zeros((rpt, 16), jnp.float32)
    ones16 = jnp.ones((LK, 16), jnp.float32)

    seg = _seg_rows(nrows, h_dim, ch)
    deg_parts = _deg_rows(nrows, 16, ch)(dst_r, ones16, zdeg)

    h0 = _tc(_enc_body, jax.ShapeDtypeStruct((n, h_dim), jnp.float32))(
        x, W_ne, b_ne)

    dis, hw0 = _tc(_pre0_body,
                   (jax.ShapeDtypeStruct((n, 1), jnp.float32),
                    jax.ShapeDtypeStruct((n, h_dim), jnp.float32)),
                   n=n)(deg_parts, h0, Wc0)

    s0 = seg(hw0, src_r, dst_r, zrows)
    h1, hw1 = _tc(_mid_body,
                  (jax.ShapeDtypeStruct((n, h_dim), jnp.float32),
                   jax.ShapeDtypeStruct((n, h_dim), jnp.float32)),
                  n=n, residual=False)(s0, hw0, dis, h0, bc0, g0, be0, Wc1)

    s1 = seg(hw1, src_r, dst_r, zrows)
    h2, hw2 = _tc(_mid_body,
                  (jax.ShapeDtypeStruct((n, h_dim), jnp.float32),
                   jax.ShapeDtypeStruct((n, h_dim), jnp.float32)),
                  n=n, residual=True)(s1, hw1, dis, h1, bc1, g1, be1, Wc2)

    s2 = seg(hw2, src_r, dst_r, zrows)
    out = _tc(_head_body, jax.ShapeDtypeStruct((n, 2), jnp.float32), n=n)(
        s2, hw2, dis, h2, bc2, g2, be2,
        Wa1, ba1, Wa2, ba2, Wk1, bk1, Wk2, bk2)
    return out
